# Initial kernel scaffold; baseline (speedup 1.0000x reference)
import jax
import jax.numpy as jnp
from jax.experimental import pallas as pl


def kernel(input_ids, position_ids, token_table, pos_table):
    raise NotImplementedError("write your pallas kernel here")


# speedup vs baseline: 1.2660x; 1.2660x over previous
import functools

import jax
import jax.numpy as jnp
from jax import lax
from jax.experimental import pallas as pl
from jax.experimental.pallas import tpu as pltpu
from jax.experimental.pallas import tpu_sc as plsc

BATCH = 4
SEQ = 2048
HIDDEN = 2048
TOKENS = BATCH * SEQ

NUM_CORES = 2
NUM_SUBCORES = 16
NUM_WORKERS = NUM_CORES * NUM_SUBCORES
LANES = 16

TOK_PER_WORKER = TOKENS // NUM_WORKERS
CHUNK = 16
NUM_CHUNKS = TOK_PER_WORKER // CHUNK
VECS_PER_CHUNK = CHUNK * HIDDEN // LANES
UNROLL = 8

_mesh = plsc.VectorSubcoreMesh(
    core_axis_name="c",
    subcore_axis_name="s",
    num_cores=NUM_CORES,
    num_subcores=NUM_SUBCORES,
)


@functools.partial(
    pl.kernel,
    out_type=jax.ShapeDtypeStruct((TOKENS, HIDDEN), jnp.float32),
    mesh=_mesh,
    scratch_types=[
        pltpu.VMEM((TOK_PER_WORKER,), jnp.int32),
        pltpu.VMEM((TOK_PER_WORKER,), jnp.int32),
        pltpu.VMEM((CHUNK, HIDDEN), jnp.float32),
        pltpu.VMEM((CHUNK, HIDDEN), jnp.float32),
        pltpu.SemaphoreType.DMA,
        pltpu.SemaphoreType.DMA,
    ],
)
def _embed_kernel(ids_hbm, pos_hbm, tok_tab, pos_tab, out_hbm,
                  ids_v, pids_v, tbuf, pbuf, sem_t, sem_p):
    wid = lax.axis_index("s") * NUM_CORES + lax.axis_index("c")
    base = wid * TOK_PER_WORKER
    pltpu.sync_copy(ids_hbm.at[pl.ds(base, TOK_PER_WORKER)], ids_v)
    pltpu.sync_copy(pos_hbm.at[pl.ds(base, TOK_PER_WORKER)], pids_v)

    def chunk_body(ch, carry):
        off = ch * CHUNK
        cp_t = pltpu.async_copy(
            tok_tab.at[ids_v.at[pl.ds(off, CHUNK)]], tbuf, sem_t)
        cp_p = pltpu.async_copy(
            pos_tab.at[pids_v.at[pl.ds(off, CHUNK)]], pbuf, sem_p)
        cp_t.wait()
        cp_p.wait()

        def add_body(j, c2):
            for u in range(UNROLL):
                v = j * UNROLL + u
                r = v // (HIDDEN // LANES)
                col = (v % (HIDDEN // LANES)) * LANES
                x = pbuf[r, pl.ds(col, LANES)]
                plsc.addupdate(tbuf.at[r, pl.ds(col, LANES)], x)
            return c2

        lax.fori_loop(0, VECS_PER_CHUNK // UNROLL, add_body, 0)
        pltpu.sync_copy(tbuf, out_hbm.at[pl.ds(base + off, CHUNK)])
        return carry

    lax.fori_loop(0, NUM_CHUNKS, chunk_body, 0)


def kernel(input_ids, position_ids, token_table, pos_table):
    ids = input_ids.reshape(TOKENS).astype(jnp.int32)
    pos = position_ids.reshape(TOKENS).astype(jnp.int32)
    out = _embed_kernel(ids, pos, token_table, pos_table)
    return out.reshape(BATCH, SEQ, HIDDEN)


# speedup vs baseline: 1.8480x; 1.4597x over previous
import functools

import jax
import jax.numpy as jnp
from jax import lax
from jax.experimental import pallas as pl
from jax.experimental.pallas import tpu as pltpu
from jax.experimental.pallas import tpu_sc as plsc

BATCH = 4
SEQ = 2048
HIDDEN = 2048
TOKENS = BATCH * SEQ

NUM_CORES = 2
NUM_SUBCORES = 16
NUM_WORKERS = NUM_CORES * NUM_SUBCORES
LANES = 16

TOK_PER_WORKER = TOKENS // NUM_WORKERS
CHUNK = 8
NUM_CHUNKS = TOK_PER_WORKER // CHUNK
VECS_PER_CHUNK = CHUNK * HIDDEN // LANES
UNROLL = 8
NBUF = 2

_mesh = plsc.VectorSubcoreMesh(
    core_axis_name="c",
    subcore_axis_name="s",
    num_cores=NUM_CORES,
    num_subcores=NUM_SUBCORES,
)


@functools.partial(
    pl.kernel,
    out_type=jax.ShapeDtypeStruct((TOKENS, HIDDEN), jnp.float32),
    mesh=_mesh,
    scratch_types=[
        pltpu.VMEM((TOK_PER_WORKER,), jnp.int32),
        pltpu.VMEM((TOK_PER_WORKER,), jnp.int32),
        [pltpu.VMEM((CHUNK, HIDDEN), jnp.float32) for _ in range(NBUF)],
        [pltpu.VMEM((CHUNK, HIDDEN), jnp.float32) for _ in range(NBUF)],
        [pltpu.VMEM((CHUNK, HIDDEN), jnp.float32) for _ in range(NBUF)],
        [pltpu.SemaphoreType.DMA for _ in range(NBUF)],
        [pltpu.SemaphoreType.DMA for _ in range(NBUF)],
        [pltpu.SemaphoreType.DMA for _ in range(NBUF)],
    ],
)
def _embed_kernel(ids_hbm, pos_hbm, tok_tab, pos_tab, out_hbm,
                  ids_v, pids_v, tbufs, pbufs, obufs, sems_t, sems_p, sems_w):
    wid = lax.axis_index("s") * NUM_CORES + lax.axis_index("c")
    base = wid * TOK_PER_WORKER
    pltpu.sync_copy(ids_hbm.at[pl.ds(base, TOK_PER_WORKER)], ids_v)
    pltpu.sync_copy(pos_hbm.at[pl.ds(base, TOK_PER_WORKER)], pids_v)

    def start_gathers(b, ch):
        off = ch * CHUNK
        pltpu.async_copy(
            tok_tab.at[ids_v.at[pl.ds(off, CHUNK)]], tbufs[b], sems_t[b])
        pltpu.async_copy(
            pos_tab.at[pids_v.at[pl.ds(off, CHUNK)]], pbufs[b], sems_p[b])

    def wait_gathers(b):
        pltpu.make_async_copy(
            tok_tab.at[ids_v.at[pl.ds(0, CHUNK)]], tbufs[b], sems_t[b]).wait()
        pltpu.make_async_copy(
            pos_tab.at[pids_v.at[pl.ds(0, CHUNK)]], pbufs[b], sems_p[b]).wait()

    def wait_write(b):
        pltpu.make_async_copy(
            obufs[b], out_hbm.at[pl.ds(base, CHUNK)], sems_w[b]).wait()

    for b in range(NBUF):
        start_gathers(b, b)

    def outer(i, carry):
        g = i * NBUF
        for b in range(NBUF):
            ch = g + b
            wait_gathers(b)

            @pl.when(ch >= NBUF)
            def _():
                wait_write(b)

            def add_body(j, c2):
                for u in range(UNROLL):
                    v = j * UNROLL + u
                    r = v // (HIDDEN // LANES)
                    col = (v % (HIDDEN // LANES)) * LANES
                    obufs[b][r, pl.ds(col, LANES)] = (
                        tbufs[b][r, pl.ds(col, LANES)]
                        + pbufs[b][r, pl.ds(col, LANES)])
                return c2

            lax.fori_loop(0, VECS_PER_CHUNK // UNROLL, add_body, 0,
                          unroll=False)

            pltpu.async_copy(
                obufs[b], out_hbm.at[pl.ds(base + ch * CHUNK, CHUNK)],
                sems_w[b])

            @pl.when(ch + NBUF < NUM_CHUNKS)
            def _():
                start_gathers(b, ch + NBUF)
        return carry

    lax.fori_loop(0, NUM_CHUNKS // NBUF, outer, 0)

    for b in range(NBUF):
        wait_write(b)


def kernel(input_ids, position_ids, token_table, pos_table):
    ids = input_ids.reshape(TOKENS).astype(jnp.int32)
    pos = position_ids.reshape(TOKENS).astype(jnp.int32)
    out = _embed_kernel(ids, pos, token_table, pos_table)
    return out.reshape(BATCH, SEQ, HIDDEN)
